# baseline (speedup 1.0000x reference)
import functools

import jax
import jax.numpy as jnp
from jax import lax
from jax.experimental import pallas as pl
from jax.experimental.pallas import tpu as pltpu
from jax.experimental.pallas import tpu_sc as plsc

NU = 10000
NI = 10000
E = 320000
R = 5
F = 128
H = 128
O = 64
NB = 16384
BAS = 2

NC = 2
NS = 16
L = 16

DEG_PAD = 10240
ROWS_PER_TILE = DEG_PAD // NS
ET = E // NS
CHUNK = 128
NCH = 160
NCHG = NCH + 2
ETP = NCH * CHUNK
PB = NB // NS
PCH = PB // CHUNK
HR = 5120
ACC_ROWS = HR + 128
ACU = ACC_ROWS // 64


def _mesh():
  return plsc.VectorSubcoreMesh(core_axis_name="c", subcore_axis_name="s",
                                num_cores=NC, num_subcores=NS)


_SC_PARAMS = pltpu.CompilerParams(needs_layout_passes=False)


def _sca_body(edge_hbm, deg_hbm, hist, idxbuf, mbuf, shist2):
  c = lax.axis_index("c")
  s = lax.axis_index("s")
  zero16 = jnp.zeros((L,), jnp.float32)
  ones16 = jnp.full((L,), 1.0, jnp.float32)

  def zero_step(i, _):
    hist[pl.ds(i * L, L)] = zero16
    return 0
  lax.fori_loop(0, DEG_PAD // L, zero_step, 0)
  pltpu.sync_copy(edge_hbm.at[pl.ds(c * E + s * ET, ET)], idxbuf)

  def acc_step(i, _):
    idx = idxbuf[pl.ds(i * L, L)]
    plsc.addupdate_scatter(hist, [idx], ones16)
    return 0
  lax.fori_loop(0, ET // L, acc_step, 0)

  pltpu.sync_copy(hist, shist2.at[pl.ds(s * DEG_PAD, DEG_PAD)])
  plsc.subcore_barrier()
  for r in range(NS):
    pltpu.sync_copy(
        shist2.at[pl.ds(r * DEG_PAD + s * ROWS_PER_TILE, ROWS_PER_TILE)],
        mbuf.at[r])

  def red_step(v, _):
    acc = zero16
    for r in range(NS):
      acc = acc + mbuf[r, pl.ds(v * L, L)]
    hist[pl.ds(v * L, L)] = acc
    return 0
  lax.fori_loop(0, ROWS_PER_TILE // L, red_step, 0)
  pltpu.sync_copy(
      hist.at[pl.ds(0, ROWS_PER_TILE)],
      deg_hbm.at[pl.ds(c * DEG_PAD + s * ROWS_PER_TILE, ROWS_PER_TILE)])


def _run_sca(edge_flat):
  f = pl.kernel(
      _sca_body,
      out_type=jax.ShapeDtypeStruct((2 * DEG_PAD,), jnp.float32),
      mesh=_mesh(),
      compiler_params=_SC_PARAMS,
      scratch_types=[
          pltpu.VMEM((DEG_PAD,), jnp.float32),
          pltpu.VMEM((ET,), jnp.int32),
          pltpu.VMEM((NS, ROWS_PER_TILE), jnp.float32),
          pltpu.VMEM_SHARED((NS * DEG_PAD,), jnp.float32),
      ],
  )
  return f(edge_flat).reshape(2, DEG_PAD)


def _scb_body(tabs_hbm, gidx_hbm, sidx0_hbm, sidx1_hbm, out_hbm,
              gbuf, sbuf, rb0, rb1, rb2, accum,
              g0, g1, g2, s0, s1, s2):
  rbufs = (rb0, rb1)
  gsem = (g0, g1)
  ssem = (s0, s1)
  c = lax.axis_index("c")
  s = lax.axis_index("s")
  zero16 = jnp.zeros((L,), jnp.float32)

  def zero_rb2():
    def zs(k, _):
      rb2[k // (H // L), pl.ds((k % (H // L)) * L, L)] = zero16
      return 0
    lax.fori_loop(0, 64 * (H // L), zs, 0)

  def zero_accum():
    for k in range(6):
      m = s + NS * k
      @pl.when(m < ACU)
      def _():
        pltpu.sync_copy(rb2, accum.at[pl.ds(m * 64, 64)])

  def copy_out(p):
    for k in range(6):
      m = s + NS * k
      @pl.when(m < ACU)
      def _():
        pltpu.sync_copy(accum.at[pl.ds(m * 64, 64)], rb2)
        pltpu.sync_copy(rb2, out_hbm.at[c, p, pl.ds(m * 64, 64)])
    zero_rb2()

  def start_g(j, b):
    pltpu.async_copy(tabs_hbm.at[gbuf.at[pl.ds(j * CHUNK, CHUNK)]],
                     rbufs[b], gsem[b])

  def wait_g(b):
    pltpu.make_async_copy(tabs_hbm.at[gbuf.at[pl.ds(0, CHUNK)]],
                          rbufs[b], gsem[b]).wait()

  def start_s(j, b):
    pltpu.async_copy(rbufs[b], accum.at[sbuf.at[j]], ssem[b], add=True)

  def wait_s(b):
    pltpu.make_async_copy(rbufs[b], accum.at[sbuf.at[0]], ssem[b]).wait()

  def run_pass(sidx_hbm):
    pltpu.sync_copy(sidx_hbm.at[c, s], sbuf)
    start_g(0, 0)
    start_g(1, 1)
    wait_g(0)
    start_s(0, 0)

    def body(k, _):
      j = 2 * k + 1
      wait_s(0); start_g(j + 1, 0); wait_g(1); start_s(j, 1)
      wait_s(1); start_g(j + 2, 1); wait_g(0); start_s(j + 1, 0)
      return 0
    lax.fori_loop(0, (NCH - 2) // 2, body, 0)
    wait_s(0); start_g(NCH, 0); wait_g(1); start_s(NCH - 1, 1)
    wait_s(1)
    wait_g(0)

  zero_rb2()
  zero_accum()
  pltpu.sync_copy(gidx_hbm.at[pl.ds((c * NS + s) * (NCHG * CHUNK), NCHG * CHUNK)],
                  gbuf)
  plsc.subcore_barrier()
  run_pass(sidx0_hbm)
  plsc.subcore_barrier()
  copy_out(0)
  plsc.subcore_barrier()
  zero_accum()
  plsc.subcore_barrier()
  run_pass(sidx1_hbm)
  plsc.subcore_barrier()
  copy_out(1)


def _run_scb(tabs, gidx, sidx0, sidx1):
  f = pl.kernel(
      _scb_body,
      out_type=jax.ShapeDtypeStruct((2, 2, ACC_ROWS, H), jnp.float32),
      mesh=_mesh(),
      compiler_params=_SC_PARAMS,
      scratch_types=[
          pltpu.VMEM((NCHG * CHUNK,), jnp.int32),
          pltpu.VMEM((NCH, CHUNK), jnp.int32),
          pltpu.VMEM((CHUNK, H), jnp.float32),
          pltpu.VMEM((CHUNK, H), jnp.float32),
          pltpu.VMEM((64, H), jnp.float32),
          pltpu.VMEM_SHARED((ACC_ROWS, H), jnp.float32),
          pltpu.SemaphoreType.DMA,
          pltpu.SemaphoreType.DMA,
          pltpu.SemaphoreType.DMA,
          pltpu.SemaphoreType.DMA,
          pltpu.SemaphoreType.DMA,
          pltpu.SemaphoreType.DMA,
      ],
  )
  return f(tabs, gidx, sidx0, sidx1)


def _scc_body(tab_hbm, pidx_hbm, out_hbm, ibuf, rbuf, sem):
  c = lax.axis_index("c")
  s = lax.axis_index("s")
  pltpu.sync_copy(pidx_hbm.at[c, s], ibuf)
  for j in range(PCH):
    pltpu.async_copy(tab_hbm.at[ibuf.at[j]], rbuf, sem).wait()
    pltpu.sync_copy(rbuf, out_hbm.at[c, pl.ds(s * PB + j * CHUNK, CHUNK)])


def _run_scc(ptab, pidx):
  f = pl.kernel(
      _scc_body,
      out_type=jax.ShapeDtypeStruct((2, NB, H), jnp.float32),
      mesh=_mesh(),
      compiler_params=_SC_PARAMS,
      scratch_types=[
          pltpu.VMEM((PCH, CHUNK), jnp.int32),
          pltpu.VMEM((CHUNK, H), jnp.float32),
          pltpu.SemaphoreType.DMA,
      ],
  )
  return f(ptab, pidx)


def _tc1_body(feat_ref, deg_ref, w_ref, out_ref):
  scale = lax.rsqrt(jnp.maximum(deg_ref[0], 1.0))
  x = feat_ref[0] * scale
  out_ref[0, 0] = jnp.dot(x, w_ref[0], preferred_element_type=jnp.float32)


def _run_tc1(feats, degs3, W):
  return pl.pallas_call(
      _tc1_body,
      grid=(2, R),
      in_specs=[
          pl.BlockSpec((1, NU, F), lambda c, r: (c, 0, 0)),
          pl.BlockSpec((1, NU, 1), lambda c, r: (c, 0, 0)),
          pl.BlockSpec((1, F, H), lambda c, r: (r, 0, 0)),
      ],
      out_specs=pl.BlockSpec((1, 1, NU, H), lambda c, r: (c, r, 0, 0)),
      out_shape=jax.ShapeDtypeStruct((2, R, NU, H), jnp.float32),
  )(feats, degs3, W)


def _tc2_body(agg_ref, deg_ref, fcw_ref, fcb_ref, out_ref):
  scale = lax.rsqrt(jnp.maximum(deg_ref[0], 1.0))
  t = agg_ref[0, :NU, :] * scale
  a = jnp.maximum(t, 0.1 * t)
  out_ref[0] = jnp.dot(a, fcw_ref[0], preferred_element_type=jnp.float32) + fcb_ref[0]


def _run_tc2(agg, degs3, fcw, fcb):
  return pl.pallas_call(
      _tc2_body,
      grid=(2,),
      in_specs=[
          pl.BlockSpec((1, DEG_PAD, H), lambda c: (c, 0, 0)),
          pl.BlockSpec((1, NU, 1), lambda c: (1 - c, 0, 0)),
          pl.BlockSpec((1, H, O), lambda c: (c, 0, 0)),
          pl.BlockSpec((1, 1, O), lambda c: (c, 0, 0)),
      ],
      out_specs=pl.BlockSpec((1, NU, O), lambda c: (c, 0, 0)),
      out_shape=jax.ShapeDtypeStruct((2, NU, O), jnp.float32),
  )(agg, degs3, fcw, fcb)


def _tc3_body(u_ref, v_ref, ps_ref, comb_ref, out_ref):
  u = u_ref[...]
  v = v_ref[...]
  s0 = jnp.sum(jnp.dot(u, ps_ref[0], preferred_element_type=jnp.float32) * v,
               axis=1, keepdims=True)
  s1 = jnp.sum(jnp.dot(u, ps_ref[1], preferred_element_type=jnp.float32) * v,
               axis=1, keepdims=True)
  out_ref[...] = s0 * comb_ref[0:1, :] + s1 * comb_ref[1:2, :]


def _run_tc3(u, v, Ps, combine):
  return pl.pallas_call(
      _tc3_body,
      out_shape=jax.ShapeDtypeStruct((NB, R), jnp.float32),
  )(u, v, Ps, combine)


def _pad_idx(x, fill, nch=NCH):
  x = x.reshape(NS, ET)
  x = jnp.pad(x, ((0, 0), (0, nch * CHUNK - ET)), constant_values=fill)
  return x.reshape(NS, nch, CHUNK)


def kernel(ufeat, ifeat, W, ufc_W, ufc_b, ifc_W, ifc_b, Ps, combine,
           edge_index, etypes, head_id, tail_id):
  edge_index = edge_index.astype(jnp.int32)
  etypes = etypes.astype(jnp.int32)
  src = edge_index[0]
  dst = edge_index[1]

  deg = _run_sca(edge_index.reshape(-1))
  degs3 = deg[:, :NU, None]

  feats = jnp.stack([ufeat, ifeat])
  tabs = _run_tc1(feats, degs3, W).reshape(2 * R * NU, H)

  gu = etypes * NU + src
  gi = R * NU + etypes * NI + dst
  gidx = jnp.stack([_pad_idx(gu, 0, NCHG), _pad_idx(gi, 0, NCHG)]).reshape(-1)
  sraw = jnp.stack([_pad_idx(dst, NU), _pad_idx(src, NU)])
  sidx0 = jnp.where(sraw < HR, sraw, HR)
  sidx1 = jnp.where(sraw >= HR, sraw - HR, HR)

  outb = _run_scb(tabs, gidx, sidx0, sidx1)
  agg = jnp.concatenate([outb[:, 0, :HR], outb[:, 1, :HR]], axis=1)

  fcw = jnp.stack([ifc_W, ufc_W])
  fcb = jnp.stack([ifc_b, ufc_b]).reshape(2, 1, O)
  outs2 = _run_tc2(agg, degs3, fcw, fcb)

  ptab = jnp.concatenate([outs2[0], outs2[1]], axis=1)
  hidx = head_id.astype(jnp.int32).reshape(NS, PCH, CHUNK)
  tidx = tail_id.astype(jnp.int32).reshape(NS, PCH, CHUNK)
  pidx = jnp.stack([hidx, tidx])
  uv = _run_scc(ptab, pidx)

  return _run_tc3(uv[0, :, O:], uv[1, :, :O], Ps, combine)


# speedup vs baseline: 1.0984x; 1.0984x over previous
import functools

import jax
import jax.numpy as jnp
from jax import lax
from jax.experimental import pallas as pl
from jax.experimental.pallas import tpu as pltpu
from jax.experimental.pallas import tpu_sc as plsc

NU = 10000
NI = 10000
E = 320000
R = 5
F = 128
H = 128
O = 64
NB = 16384
BAS = 2

NC = 2
NS = 16
L = 16

DEG_PAD = 10240
ROWS_PER_TILE = DEG_PAD // NS
ET = E // NS
CHUNK = 128
GCH = 256
NCH = 160
NG = 80
ETP = NCH * CHUNK
PB = NB // NS
PCH = PB // CHUNK
HR = 5120
ACC_ROWS = HR + 128
ACU = ACC_ROWS // 64


def _mesh():
  return plsc.VectorSubcoreMesh(core_axis_name="c", subcore_axis_name="s",
                                num_cores=NC, num_subcores=NS)


_SC_PARAMS = pltpu.CompilerParams(needs_layout_passes=False)


def _sca_body(edge_hbm, deg_hbm, hist, idxbuf, mbuf, shist2):
  c = lax.axis_index("c")
  s = lax.axis_index("s")
  zero16 = jnp.zeros((L,), jnp.float32)
  ones16 = jnp.full((L,), 1.0, jnp.float32)

  def zero_step(i, _):
    hist[pl.ds(i * L, L)] = zero16
    return 0
  lax.fori_loop(0, DEG_PAD // L, zero_step, 0)
  pltpu.sync_copy(edge_hbm.at[pl.ds(c * E + s * ET, ET)], idxbuf)

  def acc_step(i, _):
    idx = idxbuf[pl.ds(i * L, L)]
    plsc.addupdate_scatter(hist, [idx], ones16)
    return 0
  lax.fori_loop(0, ET // L, acc_step, 0)

  pltpu.sync_copy(hist, shist2.at[pl.ds(s * DEG_PAD, DEG_PAD)])
  plsc.subcore_barrier()
  for r in range(NS):
    pltpu.sync_copy(
        shist2.at[pl.ds(r * DEG_PAD + s * ROWS_PER_TILE, ROWS_PER_TILE)],
        mbuf.at[r])

  def red_step(v, _):
    acc = zero16
    for r in range(NS):
      acc = acc + mbuf[r, pl.ds(v * L, L)]
    hist[pl.ds(v * L, L)] = acc
    return 0
  lax.fori_loop(0, ROWS_PER_TILE // L, red_step, 0)
  pltpu.sync_copy(
      hist.at[pl.ds(0, ROWS_PER_TILE)],
      deg_hbm.at[pl.ds(c * DEG_PAD + s * ROWS_PER_TILE, ROWS_PER_TILE)])


def _run_sca(edge_flat):
  f = pl.kernel(
      _sca_body,
      out_type=jax.ShapeDtypeStruct((2 * DEG_PAD,), jnp.float32),
      mesh=_mesh(),
      compiler_params=_SC_PARAMS,
      scratch_types=[
          pltpu.VMEM((DEG_PAD,), jnp.float32),
          pltpu.VMEM((ET,), jnp.int32),
          pltpu.VMEM((NS, ROWS_PER_TILE), jnp.float32),
          pltpu.VMEM_SHARED((NS * DEG_PAD,), jnp.float32),
      ],
  )
  return f(edge_flat).reshape(2, DEG_PAD)


def _scb_body(tabs_hbm, gidx_hbm, sidx0_hbm, sidx1_hbm, out_hbm,
              gbuf, sbuf, rb0, rb2, accum, g0):
  c = lax.axis_index("c")
  s = lax.axis_index("s")
  zero16 = jnp.zeros((L,), jnp.float32)

  def zero_rb2():
    def zs(k, _):
      rb2[k // (H // L), pl.ds((k % (H // L)) * L, L)] = zero16
      return 0
    lax.fori_loop(0, 64 * (H // L), zs, 0)

  def zero_accum():
    for k in range(6):
      m = s + NS * k
      @pl.when(m < ACU)
      def _():
        pltpu.sync_copy(rb2, accum.at[pl.ds(m * 64, 64)])

  def copy_out(p):
    for k in range(6):
      m = s + NS * k
      @pl.when(m < ACU)
      def _():
        pltpu.sync_copy(accum.at[pl.ds(m * 64, 64)], rb2)
        pltpu.sync_copy(rb2, out_hbm.at[c, p, pl.ds(m * 64, 64)])
    zero_rb2()

  def run_pass(sidx_hbm):
    pltpu.sync_copy(sidx_hbm.at[c, s], sbuf)

    def step(j, _):
      pltpu.async_copy(tabs_hbm.at[gbuf.at[pl.ds(j * GCH, GCH)]], rb0, g0).wait()
      pltpu.sync_copy(rb0.at[pl.ds(0, CHUNK)], accum.at[sbuf.at[2 * j]], add=True)
      pltpu.sync_copy(rb0.at[pl.ds(CHUNK, CHUNK)],
                      accum.at[sbuf.at[2 * j + 1]], add=True)
      return 0
    lax.fori_loop(0, NG, step, 0)

  zero_rb2()
  zero_accum()
  pltpu.sync_copy(gidx_hbm.at[pl.ds((c * NS + s) * ETP, ETP)], gbuf)
  plsc.subcore_barrier()
  run_pass(sidx0_hbm)
  plsc.subcore_barrier()
  copy_out(0)
  plsc.subcore_barrier()
  zero_accum()
  plsc.subcore_barrier()
  run_pass(sidx1_hbm)
  plsc.subcore_barrier()
  copy_out(1)


def _run_scb(tabs, gidx, sidx0, sidx1):
  f = pl.kernel(
      _scb_body,
      out_type=jax.ShapeDtypeStruct((2, 2, ACC_ROWS, H), jnp.float32),
      mesh=_mesh(),
      compiler_params=_SC_PARAMS,
      scratch_types=[
          pltpu.VMEM((ETP,), jnp.int32),
          pltpu.VMEM((NCH, CHUNK), jnp.int32),
          pltpu.VMEM((GCH, H), jnp.float32),
          pltpu.VMEM((64, H), jnp.float32),
          pltpu.VMEM_SHARED((ACC_ROWS, H), jnp.float32),
          pltpu.SemaphoreType.DMA,
      ],
  )
  return f(tabs, gidx, sidx0, sidx1)


def _scc_body(tab_hbm, pidx_hbm, out_hbm, ibuf, rbuf, sem):
  c = lax.axis_index("c")
  s = lax.axis_index("s")
  pltpu.sync_copy(pidx_hbm.at[c, s], ibuf)
  for j in range(PCH):
    pltpu.async_copy(tab_hbm.at[ibuf.at[j]], rbuf, sem).wait()
    pltpu.sync_copy(rbuf, out_hbm.at[c, pl.ds(s * PB + j * CHUNK, CHUNK)])


def _run_scc(ptab, pidx):
  f = pl.kernel(
      _scc_body,
      out_type=jax.ShapeDtypeStruct((2, NB, H), jnp.float32),
      mesh=_mesh(),
      compiler_params=_SC_PARAMS,
      scratch_types=[
          pltpu.VMEM((PCH, CHUNK), jnp.int32),
          pltpu.VMEM((CHUNK, H), jnp.float32),
          pltpu.SemaphoreType.DMA,
      ],
  )
  return f(ptab, pidx)


def _tc1_body(feat_ref, deg_ref, w_ref, out_ref):
  scale = lax.rsqrt(jnp.maximum(deg_ref[0], 1.0))
  x = feat_ref[0] * scale
  out_ref[0, 0] = jnp.dot(x, w_ref[0], preferred_element_type=jnp.float32)


def _run_tc1(feats, degs3, W):
  return pl.pallas_call(
      _tc1_body,
      grid=(2, R),
      in_specs=[
          pl.BlockSpec((1, NU, F), lambda c, r: (c, 0, 0)),
          pl.BlockSpec((1, NU, 1), lambda c, r: (c, 0, 0)),
          pl.BlockSpec((1, F, H), lambda c, r: (r, 0, 0)),
      ],
      out_specs=pl.BlockSpec((1, 1, NU, H), lambda c, r: (c, r, 0, 0)),
      out_shape=jax.ShapeDtypeStruct((2, R, NU, H), jnp.float32),
  )(feats, degs3, W)


def _tc2_body(agg_ref, deg_ref, fcw_ref, fcb_ref, out_ref):
  scale = lax.rsqrt(jnp.maximum(deg_ref[0], 1.0))
  t = agg_ref[0, :NU, :] * scale
  a = jnp.maximum(t, 0.1 * t)
  out_ref[0] = jnp.dot(a, fcw_ref[0], preferred_element_type=jnp.float32) + fcb_ref[0]


def _run_tc2(agg, degs3, fcw, fcb):
  return pl.pallas_call(
      _tc2_body,
      grid=(2,),
      in_specs=[
          pl.BlockSpec((1, DEG_PAD, H), lambda c: (c, 0, 0)),
          pl.BlockSpec((1, NU, 1), lambda c: (1 - c, 0, 0)),
          pl.BlockSpec((1, H, O), lambda c: (c, 0, 0)),
          pl.BlockSpec((1, 1, O), lambda c: (c, 0, 0)),
      ],
      out_specs=pl.BlockSpec((1, NU, O), lambda c: (c, 0, 0)),
      out_shape=jax.ShapeDtypeStruct((2, NU, O), jnp.float32),
  )(agg, degs3, fcw, fcb)


def _tc3_body(u_ref, v_ref, ps_ref, comb_ref, out_ref):
  u = u_ref[...]
  v = v_ref[...]
  s0 = jnp.sum(jnp.dot(u, ps_ref[0], preferred_element_type=jnp.float32) * v,
               axis=1, keepdims=True)
  s1 = jnp.sum(jnp.dot(u, ps_ref[1], preferred_element_type=jnp.float32) * v,
               axis=1, keepdims=True)
  out_ref[...] = s0 * comb_ref[0:1, :] + s1 * comb_ref[1:2, :]


def _run_tc3(u, v, Ps, combine):
  return pl.pallas_call(
      _tc3_body,
      out_shape=jax.ShapeDtypeStruct((NB, R), jnp.float32),
  )(u, v, Ps, combine)


def _pad_idx(x, fill, nch=NCH):
  x = x.reshape(NS, ET)
  x = jnp.pad(x, ((0, 0), (0, nch * CHUNK - ET)), constant_values=fill)
  return x.reshape(NS, nch, CHUNK)


def kernel(ufeat, ifeat, W, ufc_W, ufc_b, ifc_W, ifc_b, Ps, combine,
           edge_index, etypes, head_id, tail_id):
  edge_index = edge_index.astype(jnp.int32)
  etypes = etypes.astype(jnp.int32)
  src = edge_index[0]
  dst = edge_index[1]

  deg = _run_sca(edge_index.reshape(-1))
  degs3 = deg[:, :NU, None]

  feats = jnp.stack([ufeat, ifeat])
  tabs = _run_tc1(feats, degs3, W).reshape(2 * R * NU, H)

  gu = etypes * NU + src
  gi = R * NU + etypes * NI + dst
  gidx = jnp.stack([_pad_idx(gu, 0), _pad_idx(gi, 0)]).reshape(-1)
  sraw = jnp.stack([_pad_idx(dst, NU), _pad_idx(src, NU)])
  sidx0 = jnp.where(sraw < HR, sraw, HR)
  sidx1 = jnp.where(sraw >= HR, sraw - HR, HR)

  outb = _run_scb(tabs, gidx, sidx0, sidx1)
  agg = jnp.concatenate([outb[:, 0, :HR], outb[:, 1, :HR]], axis=1)

  fcw = jnp.stack([ifc_W, ufc_W])
  fcb = jnp.stack([ifc_b, ufc_b]).reshape(2, 1, O)
  outs2 = _run_tc2(agg, degs3, fcw, fcb)

  ptab = jnp.concatenate([outs2[0], outs2[1]], axis=1)
  hidx = head_id.astype(jnp.int32).reshape(NS, PCH, CHUNK)
  tidx = tail_id.astype(jnp.int32).reshape(NS, PCH, CHUNK)
  pidx = jnp.stack([hidx, tidx])
  uv = _run_scc(ptab, pidx)

  return _run_tc3(uv[0, :, O:], uv[1, :, :O], Ps, combine)


# speedup vs baseline: 1.1060x; 1.0069x over previous
import functools

import jax
import jax.numpy as jnp
from jax import lax
from jax.experimental import pallas as pl
from jax.experimental.pallas import tpu as pltpu
from jax.experimental.pallas import tpu_sc as plsc

NU = 10000
NI = 10000
E = 320000
R = 5
F = 128
H = 128
O = 64
NB = 16384
BAS = 2

NC = 2
NS = 16
L = 16

DEG_PAD = 10240
ROWS_PER_TILE = DEG_PAD // NS
ET = E // NS
CHUNK = 128
NCH = 160
ETP = NCH * CHUNK
PB = NB // NS
PCH = PB // CHUNK
HR = 5120
ACC_ROWS = HR + 128
ACU = ACC_ROWS // 64


def _mesh():
  return plsc.VectorSubcoreMesh(core_axis_name="c", subcore_axis_name="s",
                                num_cores=NC, num_subcores=NS)


_SC_PARAMS = pltpu.CompilerParams(needs_layout_passes=False)


def _sca_body(edge_hbm, deg_hbm, hist, idxbuf, mbuf, shist2):
  c = lax.axis_index("c")
  s = lax.axis_index("s")
  zero16 = jnp.zeros((L,), jnp.float32)
  ones16 = jnp.full((L,), 1.0, jnp.float32)

  def zero_step(i, _):
    hist[pl.ds(i * L, L)] = zero16
    return 0
  lax.fori_loop(0, DEG_PAD // L, zero_step, 0)
  pltpu.sync_copy(edge_hbm.at[pl.ds(c * E + s * ET, ET)], idxbuf)

  def acc_step(i, _):
    idx = idxbuf[pl.ds(i * L, L)]
    plsc.addupdate_scatter(hist, [idx], ones16)
    return 0
  lax.fori_loop(0, ET // L, acc_step, 0)

  pltpu.sync_copy(hist, shist2.at[pl.ds(s * DEG_PAD, DEG_PAD)])
  plsc.subcore_barrier()
  for r in range(NS):
    pltpu.sync_copy(
        shist2.at[pl.ds(r * DEG_PAD + s * ROWS_PER_TILE, ROWS_PER_TILE)],
        mbuf.at[r])

  def red_step(v, _):
    acc = zero16
    for r in range(NS):
      acc = acc + mbuf[r, pl.ds(v * L, L)]
    hist[pl.ds(v * L, L)] = acc
    return 0
  lax.fori_loop(0, ROWS_PER_TILE // L, red_step, 0)
  pltpu.sync_copy(
      hist.at[pl.ds(0, ROWS_PER_TILE)],
      deg_hbm.at[pl.ds(c * DEG_PAD + s * ROWS_PER_TILE, ROWS_PER_TILE)])


def _run_sca(edge_flat):
  f = pl.kernel(
      _sca_body,
      out_type=jax.ShapeDtypeStruct((2 * DEG_PAD,), jnp.float32),
      mesh=_mesh(),
      compiler_params=_SC_PARAMS,
      scratch_types=[
          pltpu.VMEM((DEG_PAD,), jnp.float32),
          pltpu.VMEM((ET,), jnp.int32),
          pltpu.VMEM((NS, ROWS_PER_TILE), jnp.float32),
          pltpu.VMEM_SHARED((NS * DEG_PAD,), jnp.float32),
      ],
  )
  return f(edge_flat).reshape(2, DEG_PAD)


def _scb_body(tabs_hbm, gidx_hbm, sidx0_hbm, sidx1_hbm, out_hbm,
              gbuf, sbuf, rba, rbb, rb2, accum, ga, gb):
  c = lax.axis_index("c")
  s = lax.axis_index("s")
  zero16 = jnp.zeros((L,), jnp.float32)

  def zero_rb2():
    def zs(k, _):
      rb2[k // (H // L), pl.ds((k % (H // L)) * L, L)] = zero16
      return 0
    lax.fori_loop(0, 64 * (H // L), zs, 0)

  def zero_accum():
    for k in range(6):
      m = s + NS * k
      @pl.when(m < ACU)
      def _():
        pltpu.sync_copy(rb2, accum.at[pl.ds(m * 64, 64)])

  def copy_out(p):
    for k in range(6):
      m = s + NS * k
      @pl.when(m < ACU)
      def _():
        pltpu.sync_copy(accum.at[pl.ds(m * 64, 64)], rb2)
        pltpu.sync_copy(rb2, out_hbm.at[c, p, pl.ds(m * 64, 64)])
    zero_rb2()

  def run_pass(sidx_hbm):
    pltpu.sync_copy(sidx_hbm.at[c, s], sbuf)

    def step(k, _):
      j = 2 * k
      da = pltpu.async_copy(tabs_hbm.at[gbuf.at[j]], rba, ga)
      db = pltpu.async_copy(tabs_hbm.at[gbuf.at[j + 1]], rbb, gb)
      da.wait()
      pltpu.sync_copy(rba, accum.at[sbuf.at[j]], add=True)
      db.wait()
      pltpu.sync_copy(rbb, accum.at[sbuf.at[j + 1]], add=True)
      return 0
    lax.fori_loop(0, NCH // 2, step, 0)

  zero_rb2()
  zero_accum()
  pltpu.sync_copy(gidx_hbm.at[c, s], gbuf)
  plsc.subcore_barrier()
  run_pass(sidx0_hbm)
  plsc.subcore_barrier()
  copy_out(0)
  plsc.subcore_barrier()
  zero_accum()
  plsc.subcore_barrier()
  run_pass(sidx1_hbm)
  plsc.subcore_barrier()
  copy_out(1)


def _run_scb(tabs, gidx, sidx0, sidx1):
  f = pl.kernel(
      _scb_body,
      out_type=jax.ShapeDtypeStruct((2, 2, ACC_ROWS, H), jnp.float32),
      mesh=_mesh(),
      compiler_params=_SC_PARAMS,
      scratch_types=[
          pltpu.VMEM((NCH, CHUNK), jnp.int32),
          pltpu.VMEM((NCH, CHUNK), jnp.int32),
          pltpu.VMEM((CHUNK, H), jnp.float32),
          pltpu.VMEM((CHUNK, H), jnp.float32),
          pltpu.VMEM((64, H), jnp.float32),
          pltpu.VMEM_SHARED((ACC_ROWS, H), jnp.float32),
          pltpu.SemaphoreType.DMA,
          pltpu.SemaphoreType.DMA,
      ],
  )
  return f(tabs, gidx, sidx0, sidx1)


def _scc_body(tab_hbm, pidx_hbm, out_hbm, ibuf, rbuf, sem):
  c = lax.axis_index("c")
  s = lax.axis_index("s")
  pltpu.sync_copy(pidx_hbm.at[c, s], ibuf)
  for j in range(PCH):
    pltpu.async_copy(tab_hbm.at[ibuf.at[j]], rbuf, sem).wait()
    pltpu.sync_copy(rbuf, out_hbm.at[c, pl.ds(s * PB + j * CHUNK, CHUNK)])


def _run_scc(ptab, pidx):
  f = pl.kernel(
      _scc_body,
      out_type=jax.ShapeDtypeStruct((2, NB, H), jnp.float32),
      mesh=_mesh(),
      compiler_params=_SC_PARAMS,
      scratch_types=[
          pltpu.VMEM((PCH, CHUNK), jnp.int32),
          pltpu.VMEM((CHUNK, H), jnp.float32),
          pltpu.SemaphoreType.DMA,
      ],
  )
  return f(ptab, pidx)


def _tc1_body(feat_ref, deg_ref, w_ref, out_ref):
  scale = lax.rsqrt(jnp.maximum(deg_ref[0], 1.0))
  x = feat_ref[0] * scale
  out_ref[0, 0] = jnp.dot(x, w_ref[0], preferred_element_type=jnp.float32)


def _run_tc1(feats, degs3, W):
  return pl.pallas_call(
      _tc1_body,
      grid=(2, R),
      in_specs=[
          pl.BlockSpec((1, NU, F), lambda c, r: (c, 0, 0)),
          pl.BlockSpec((1, NU, 1), lambda c, r: (c, 0, 0)),
          pl.BlockSpec((1, F, H), lambda c, r: (r, 0, 0)),
      ],
      out_specs=pl.BlockSpec((1, 1, NU, H), lambda c, r: (c, r, 0, 0)),
      out_shape=jax.ShapeDtypeStruct((2, R, NU, H), jnp.float32),
  )(feats, degs3, W)


def _tc2_body(agg_ref, deg_ref, fcw_ref, fcb_ref, out_ref):
  scale = lax.rsqrt(jnp.maximum(deg_ref[0], 1.0))
  t = agg_ref[0, :NU, :] * scale
  a = jnp.maximum(t, 0.1 * t)
  out_ref[0] = jnp.dot(a, fcw_ref[0], preferred_element_type=jnp.float32) + fcb_ref[0]


def _run_tc2(agg, degs3, fcw, fcb):
  return pl.pallas_call(
      _tc2_body,
      grid=(2,),
      in_specs=[
          pl.BlockSpec((1, DEG_PAD, H), lambda c: (c, 0, 0)),
          pl.BlockSpec((1, NU, 1), lambda c: (1 - c, 0, 0)),
          pl.BlockSpec((1, H, O), lambda c: (c, 0, 0)),
          pl.BlockSpec((1, 1, O), lambda c: (c, 0, 0)),
      ],
      out_specs=pl.BlockSpec((1, NU, O), lambda c: (c, 0, 0)),
      out_shape=jax.ShapeDtypeStruct((2, NU, O), jnp.float32),
  )(agg, degs3, fcw, fcb)


def _tc3_body(u_ref, v_ref, ps_ref, comb_ref, out_ref):
  u = u_ref[...]
  v = v_ref[...]
  s0 = jnp.sum(jnp.dot(u, ps_ref[0], preferred_element_type=jnp.float32) * v,
               axis=1, keepdims=True)
  s1 = jnp.sum(jnp.dot(u, ps_ref[1], preferred_element_type=jnp.float32) * v,
               axis=1, keepdims=True)
  out_ref[...] = s0 * comb_ref[0:1, :] + s1 * comb_ref[1:2, :]


def _run_tc3(u, v, Ps, combine):
  return pl.pallas_call(
      _tc3_body,
      out_shape=jax.ShapeDtypeStruct((NB, R), jnp.float32),
  )(u, v, Ps, combine)


def _pad_idx(x, fill, nch=NCH):
  x = x.reshape(NS, ET)
  x = jnp.pad(x, ((0, 0), (0, nch * CHUNK - ET)), constant_values=fill)
  return x.reshape(NS, nch, CHUNK)


def kernel(ufeat, ifeat, W, ufc_W, ufc_b, ifc_W, ifc_b, Ps, combine,
           edge_index, etypes, head_id, tail_id):
  edge_index = edge_index.astype(jnp.int32)
  etypes = etypes.astype(jnp.int32)
  src = edge_index[0]
  dst = edge_index[1]

  deg = _run_sca(edge_index.reshape(-1))
  degs3 = deg[:, :NU, None]

  feats = jnp.stack([ufeat, ifeat])
  tabs = _run_tc1(feats, degs3, W).reshape(2 * R * NU, H)

  gu = etypes * NU + src
  gi = R * NU + etypes * NI + dst
  gidx = jnp.stack([_pad_idx(gu, 0), _pad_idx(gi, 0)])
  sraw = jnp.stack([_pad_idx(dst, NU), _pad_idx(src, NU)])
  sidx0 = jnp.where(sraw < HR, sraw, HR)
  sidx1 = jnp.where(sraw >= HR, sraw - HR, HR)

  outb = _run_scb(tabs, gidx, sidx0, sidx1)
  agg = jnp.concatenate([outb[:, 0, :HR], outb[:, 1, :HR]], axis=1)

  fcw = jnp.stack([ifc_W, ufc_W])
  fcb = jnp.stack([ifc_b, ufc_b]).reshape(2, 1, O)
  outs2 = _run_tc2(agg, degs3, fcw, fcb)

  ptab = jnp.concatenate([outs2[0], outs2[1]], axis=1)
  hidx = head_id.astype(jnp.int32).reshape(NS, PCH, CHUNK)
  tidx = tail_id.astype(jnp.int32).reshape(NS, PCH, CHUNK)
  pidx = jnp.stack([hidx, tidx])
  uv = _run_scc(ptab, pidx)

  return _run_tc3(uv[0, :, O:], uv[1, :, :O], Ps, combine)


# speedup vs baseline: 2.0389x; 1.8435x over previous
import functools

import jax
import jax.numpy as jnp
from jax import lax
from jax.experimental import pallas as pl
from jax.experimental.pallas import tpu as pltpu
from jax.experimental.pallas import tpu_sc as plsc

NU = 10000
NI = 10000
E = 320000
R = 5
F = 128
H = 128
O = 64
NB = 16384
BAS = 2

NC = 2
NS = 16
L = 16

DEG_PAD = 10240
ROWS_PER_TILE = DEG_PAD // NS
ET = E // NS
CHUNK = 128
NCH = 157
ETP = NCH * CHUNK
PB = NB // NS
PCH = PB // CHUNK
HR = 5120
ACC_ROWS = HR + 128
ACH = ACC_ROWS // CHUNK


def _mesh():
  return plsc.VectorSubcoreMesh(core_axis_name="c", subcore_axis_name="s",
                                num_cores=NC, num_subcores=NS)


_SC_PARAMS = pltpu.CompilerParams(needs_layout_passes=False)


def _sca_body(edge_hbm, deg_hbm, hist, idxbuf, mbuf, shist2):
  c = lax.axis_index("c")
  s = lax.axis_index("s")
  zero16 = jnp.zeros((L,), jnp.float32)
  ones16 = jnp.full((L,), 1.0, jnp.float32)

  def zero_step(i, _):
    hist[pl.ds(i * L, L)] = zero16
    return 0
  lax.fori_loop(0, DEG_PAD // L, zero_step, 0)
  pltpu.sync_copy(edge_hbm.at[pl.ds(c * E + s * ET, ET)], idxbuf)

  def acc_step(i, _):
    idx = idxbuf[pl.ds(i * L, L)]
    plsc.addupdate_scatter(hist, [idx], ones16)
    return 0
  lax.fori_loop(0, ET // L, acc_step, 0)

  pltpu.sync_copy(hist, shist2.at[pl.ds(s * DEG_PAD, DEG_PAD)])
  plsc.subcore_barrier()
  for r in range(NS):
    pltpu.sync_copy(
        shist2.at[pl.ds(r * DEG_PAD + s * ROWS_PER_TILE, ROWS_PER_TILE)],
        mbuf.at[r])

  def red_step(v, _):
    acc = zero16
    for r in range(NS):
      acc = acc + mbuf[r, pl.ds(v * L, L)]
    hist[pl.ds(v * L, L)] = acc
    return 0
  lax.fori_loop(0, ROWS_PER_TILE // L, red_step, 0)
  pltpu.sync_copy(
      hist.at[pl.ds(0, ROWS_PER_TILE)],
      deg_hbm.at[pl.ds(c * DEG_PAD + s * ROWS_PER_TILE, ROWS_PER_TILE)])


def _run_sca(edge_flat):
  f = pl.kernel(
      _sca_body,
      out_type=jax.ShapeDtypeStruct((2 * DEG_PAD,), jnp.float32),
      mesh=_mesh(),
      compiler_params=_SC_PARAMS,
      scratch_types=[
          pltpu.VMEM((DEG_PAD,), jnp.float32),
          pltpu.VMEM((ET,), jnp.int32),
          pltpu.VMEM((NS, ROWS_PER_TILE), jnp.float32),
          pltpu.VMEM_SHARED((NS * DEG_PAD,), jnp.float32),
      ],
  )
  return f(edge_flat).reshape(2, DEG_PAD)


def _scb_body(tabs_hbm, gidx_hbm, sidx0_hbm, sidx1_hbm, out_hbm,
              gbuf, sbuf, rbuf, zbuf, accum, sem):
  c = lax.axis_index("c")
  s = lax.axis_index("s")
  zero16 = jnp.zeros((L,), jnp.float32)

  def zero_step(k, _):
    zbuf[k // (H // L), pl.ds((k % (H // L)) * L, L)] = zero16
    return 0
  lax.fori_loop(0, CHUNK * (H // L), zero_step, 0)

  def zero_accum():
    for k in range(3):
      m = s + NS * k
      @pl.when(m < ACH)
      def _():
        pltpu.sync_copy(zbuf, accum.at[pl.ds(m * CHUNK, CHUNK)])

  def copy_out(p):
    for k in range(3):
      m = s + NS * k
      @pl.when(m < ACH)
      def _():
        pltpu.sync_copy(accum.at[pl.ds(m * CHUNK, CHUNK)], rbuf)
        pltpu.sync_copy(rbuf, out_hbm.at[c, p, pl.ds(m * CHUNK, CHUNK)])

  def run_pass(sidx_hbm):
    pltpu.sync_copy(sidx_hbm.at[c, s], sbuf)

    def step(j, _):
      pltpu.async_copy(tabs_hbm.at[gbuf.at[j]], rbuf, sem).wait()
      pltpu.sync_copy(rbuf, accum.at[sbuf.at[j]], add=True)
      return 0
    lax.fori_loop(0, NCH, step, 0)

  zero_accum()
  pltpu.sync_copy(gidx_hbm.at[c, s], gbuf)
  plsc.subcore_barrier()
  run_pass(sidx0_hbm)
  plsc.subcore_barrier()
  copy_out(0)
  plsc.subcore_barrier()
  zero_accum()
  plsc.subcore_barrier()
  run_pass(sidx1_hbm)
  plsc.subcore_barrier()
  copy_out(1)


def _run_scb(tabs, gidx, sidx0, sidx1):
  f = pl.kernel(
      _scb_body,
      out_type=jax.ShapeDtypeStruct((2, 2, ACC_ROWS, H), jnp.float32),
      mesh=_mesh(),
      compiler_params=_SC_PARAMS,
      scratch_types=[
          pltpu.VMEM((NCH, CHUNK), jnp.int32),
          pltpu.VMEM((NCH, CHUNK), jnp.int32),
          pltpu.VMEM((CHUNK, H), jnp.float32),
          pltpu.VMEM((CHUNK, H), jnp.float32),
          pltpu.VMEM_SHARED((ACC_ROWS, H), jnp.float32),
          pltpu.SemaphoreType.DMA,
      ],
  )
  return f(tabs, gidx, sidx0, sidx1)


def _scc_body(tab_hbm, pidx_hbm, out_hbm, ibuf, rbuf, sem):
  c = lax.axis_index("c")
  s = lax.axis_index("s")
  pltpu.sync_copy(pidx_hbm.at[c, s], ibuf)
  for j in range(PCH):
    pltpu.async_copy(tab_hbm.at[ibuf.at[j]], rbuf, sem).wait()
    pltpu.sync_copy(rbuf, out_hbm.at[c, pl.ds(s * PB + j * CHUNK, CHUNK)])


def _run_scc(ptab, pidx):
  f = pl.kernel(
      _scc_body,
      out_type=jax.ShapeDtypeStruct((2, NB, H), jnp.float32),
      mesh=_mesh(),
      compiler_params=_SC_PARAMS,
      scratch_types=[
          pltpu.VMEM((PCH, CHUNK), jnp.int32),
          pltpu.VMEM((CHUNK, H), jnp.float32),
          pltpu.SemaphoreType.DMA,
      ],
  )
  return f(ptab, pidx)


def _tc1_body(feat_ref, deg_ref, w_ref, out_ref):
  scale = lax.rsqrt(jnp.maximum(deg_ref[0], 1.0))
  x = feat_ref[0] * scale
  out_ref[0, 0] = jnp.dot(x, w_ref[0], preferred_element_type=jnp.float32)


def _run_tc1(feats, degs3, W):
  return pl.pallas_call(
      _tc1_body,
      grid=(2, R),
      in_specs=[
          pl.BlockSpec((1, NU, F), lambda c, r: (c, 0, 0)),
          pl.BlockSpec((1, NU, 1), lambda c, r: (c, 0, 0)),
          pl.BlockSpec((1, F, H), lambda c, r: (r, 0, 0)),
      ],
      out_specs=pl.BlockSpec((1, 1, NU, H), lambda c, r: (c, r, 0, 0)),
      out_shape=jax.ShapeDtypeStruct((2, R, NU, H), jnp.float32),
  )(feats, degs3, W)


def _tc2_body(agg_ref, deg_ref, fcw_ref, fcb_ref, out_ref):
  scale = lax.rsqrt(jnp.maximum(deg_ref[0], 1.0))
  t = agg_ref[0, :NU, :] * scale
  a = jnp.maximum(t, 0.1 * t)
  out_ref[0] = jnp.dot(a, fcw_ref[0], preferred_element_type=jnp.float32) + fcb_ref[0]


def _run_tc2(agg, degs3, fcw, fcb):
  return pl.pallas_call(
      _tc2_body,
      grid=(2,),
      in_specs=[
          pl.BlockSpec((1, DEG_PAD, H), lambda c: (c, 0, 0)),
          pl.BlockSpec((1, NU, 1), lambda c: (1 - c, 0, 0)),
          pl.BlockSpec((1, H, O), lambda c: (c, 0, 0)),
          pl.BlockSpec((1, 1, O), lambda c: (c, 0, 0)),
      ],
      out_specs=pl.BlockSpec((1, NU, O), lambda c: (c, 0, 0)),
      out_shape=jax.ShapeDtypeStruct((2, NU, O), jnp.float32),
  )(agg, degs3, fcw, fcb)


def _tc3_body(u_ref, v_ref, ps_ref, comb_ref, out_ref):
  u = u_ref[...]
  v = v_ref[...]
  s0 = jnp.sum(jnp.dot(u, ps_ref[0], preferred_element_type=jnp.float32) * v,
               axis=1, keepdims=True)
  s1 = jnp.sum(jnp.dot(u, ps_ref[1], preferred_element_type=jnp.float32) * v,
               axis=1, keepdims=True)
  out_ref[...] = s0 * comb_ref[0:1, :] + s1 * comb_ref[1:2, :]


def _run_tc3(u, v, Ps, combine):
  return pl.pallas_call(
      _tc3_body,
      out_shape=jax.ShapeDtypeStruct((NB, R), jnp.float32),
  )(u, v, Ps, combine)


def _pad_idx(x, fill, nch=NCH):
  x = x.reshape(NS, ET)
  x = jnp.pad(x, ((0, 0), (0, nch * CHUNK - ET)), constant_values=fill)
  return x.reshape(NS, nch, CHUNK)


def kernel(ufeat, ifeat, W, ufc_W, ufc_b, ifc_W, ifc_b, Ps, combine,
           edge_index, etypes, head_id, tail_id):
  edge_index = edge_index.astype(jnp.int32)
  etypes = etypes.astype(jnp.int32)
  src = edge_index[0]
  dst = edge_index[1]

  deg = _run_sca(edge_index.reshape(-1))
  degs3 = deg[:, :NU, None]

  feats = jnp.stack([ufeat, ifeat])
  tabs = _run_tc1(feats, degs3, W).reshape(2 * R * NU, H)

  gu = etypes * NU + src
  gi = R * NU + etypes * NI + dst
  gidx = jnp.stack([_pad_idx(gu, 0), _pad_idx(gi, 0)])
  sraw = jnp.stack([_pad_idx(dst, NU), _pad_idx(src, NU)])
  sidx0 = jnp.where(sraw < HR, sraw, HR)
  sidx1 = jnp.where(sraw >= HR, sraw - HR, HR)

  outb = _run_scb(tabs, gidx, sidx0, sidx1)
  agg = jnp.concatenate([outb[:, 0, :HR], outb[:, 1, :HR]], axis=1)

  fcw = jnp.stack([ifc_W, ufc_W])
  fcb = jnp.stack([ifc_b, ufc_b]).reshape(2, 1, O)
  outs2 = _run_tc2(agg, degs3, fcw, fcb)

  ptab = jnp.concatenate([outs2[0], outs2[1]], axis=1)
  hidx = head_id.astype(jnp.int32).reshape(NS, PCH, CHUNK)
  tidx = tail_id.astype(jnp.int32).reshape(NS, PCH, CHUNK)
  pidx = jnp.stack([hidx, tidx])
  uv = _run_scc(ptab, pidx)

  return _run_tc3(uv[0, :, O:], uv[1, :, :O], Ps, combine)
